# baseline (speedup 1.0000x reference)
import functools

import jax
import jax.numpy as jnp
from jax import lax
from jax.experimental import pallas as pl
from jax.experimental.pallas import tpu as pltpu
from jax.experimental.pallas import tpu_sc as plsc

_U = 1024
_V = 1024
_L = 64
_HD = 16
_N = 524288
_NC = 2
_NS = 16
_NW = _NC * _NS
_CHUNK = 512
_PER_W = _N // _NW
_STEPS = _PER_W // _CHUNK
_GSZ = 128
_NG = _CHUNK // _GSZ


def _body(h_hbm, u_hbm, v_hbm, lat_hbm, out_hbm,
          u_buf, v_buf, idx_buf, h_buf, rows_buf, sem):
  wid = lax.axis_index("s") * _NC + lax.axis_index("c")

  def step(g, carry):
    base = wid * _PER_W + g * _CHUNK
    pltpu.sync_copy(u_hbm.at[pl.ds(base, _CHUNK)], u_buf)
    pltpu.sync_copy(v_hbm.at[pl.ds(base, _CHUNK)], v_buf)
    h_cp = pltpu.async_copy(h_hbm.at[pl.ds(base, _CHUNK)], h_buf, sem)
    for j in range(_CHUNK // 16):
      uu = u_buf[pl.ds(j * 16, 16)] * float(_U)
      vv = v_buf[pl.ds(j * 16, 16)] * float(_V)
      iu = jnp.minimum(uu.astype(jnp.int32), _U - 1)
      iv = jnp.minimum(vv.astype(jnp.int32), _V - 1)
      idx_buf[pl.ds(j * 16, 16)] = iu * _V + iv
    gathers = [
        pltpu.async_copy(
            lat_hbm.at[idx_buf.at[pl.ds(k * _GSZ, _GSZ)]],
            rows_buf.at[pl.ds(k * _GSZ, _GSZ)],
            sem,
        )
        for k in range(_NG)
    ]
    h_cp.wait()
    pltpu.sync_copy(h_buf, out_hbm.at[pl.ds(base, _CHUNK), pl.ds(0, _HD)])
    for g_cp in gathers:
      g_cp.wait()
    pltpu.sync_copy(rows_buf,
                    out_hbm.at[pl.ds(base, _CHUNK), pl.ds(_HD, _L)])
    return carry

  lax.fori_loop(0, _STEPS, step, 0)


@jax.jit
def _nlb(h, u, v, lat2d):
  mesh = plsc.VectorSubcoreMesh(
      core_axis_name="c", subcore_axis_name="s",
      num_cores=_NC, num_subcores=_NS)
  return pl.kernel(
      _body,
      out_type=jax.ShapeDtypeStruct((_N, _HD + _L), jnp.float32),
      mesh=mesh,
      scratch_types=[
          pltpu.VMEM((_CHUNK,), jnp.float32),
          pltpu.VMEM((_CHUNK,), jnp.float32),
          pltpu.VMEM((_CHUNK,), jnp.int32),
          pltpu.VMEM((_CHUNK, _HD), jnp.float32),
          pltpu.VMEM((_CHUNK, _L), jnp.float32),
          pltpu.SemaphoreType.DMA,
      ],
      compiler_params=pltpu.CompilerParams(use_tc_tiling_on_sc=False),
  )(h, u, v, lat2d)


def kernel(r, h, u, v, latents):
  del r
  return _nlb(h, u, v, latents.reshape(_U * _V, _L))


# speedup vs baseline: 1.0296x; 1.0296x over previous
import functools

import jax
import jax.numpy as jnp
from jax import lax
from jax.experimental import pallas as pl
from jax.experimental.pallas import tpu as pltpu
from jax.experimental.pallas import tpu_sc as plsc

_U = 1024
_V = 1024
_L = 64
_HD = 16
_OD = _HD + _L
_N = 524288
_NC = 2
_NS = 16
_NW = _NC * _NS
_CHUNK = 512
_PER_W = _N // _NW
_STEPS = _PER_W // _CHUNK
_GSZ = 128
_NG = _CHUNK // _GSZ


def _body(h_hbm, u_hbm, v_hbm, lat_hbm, out_hbm,
          u_buf, v_buf, idx0, idx1, hb0, hb1, rb0, rb1,
          sem_uv, sem_in0, sem_in1, sem_w0, sem_w1):
  wid = lax.axis_index("s") * _NC + lax.axis_index("c")
  idx_bufs = (idx0, idx1)
  h_bufs = (hb0, hb1)
  r_bufs = (rb0, rb1)
  sem_in = (sem_in0, sem_in1)
  sem_w = (sem_w0, sem_w1)

  def stage_in(g, p):
    base = wid * _PER_W + g * _CHUNK
    u_cp = pltpu.async_copy(u_hbm.at[pl.ds(base, _CHUNK)], u_buf, sem_uv)
    v_cp = pltpu.async_copy(v_hbm.at[pl.ds(base, _CHUNK)], v_buf, sem_uv)
    u_cp.wait()
    v_cp.wait()
    for j in range(_CHUNK // 16):
      uu = u_buf[pl.ds(j * 16, 16)] * float(_U)
      vv = v_buf[pl.ds(j * 16, 16)] * float(_V)
      iu = jnp.minimum(uu.astype(jnp.int32), _U - 1)
      iv = jnp.minimum(vv.astype(jnp.int32), _V - 1)
      idx_bufs[p][pl.ds(j * 16, 16)] = iu * _V + iv
    pltpu.async_copy(h_hbm.at[pl.ds(base, _CHUNK)], h_bufs[p], sem_in[p])
    for k in range(_NG):
      pltpu.async_copy(
          lat_hbm.at[idx_bufs[p].at[pl.ds(k * _GSZ, _GSZ)]],
          r_bufs[p].at[pl.ds(k * _GSZ, _GSZ)],
          sem_in[p],
      )

  def drain_in(p):
    pltpu.make_async_copy(h_hbm.at[pl.ds(0, _CHUNK)], h_bufs[p],
                          sem_in[p]).wait()
    for k in range(_NG):
      pltpu.make_async_copy(
          h_hbm.at[pl.ds(0, _CHUNK)],
          r_bufs[p].at[pl.ds(k * _GSZ, _GSZ)],
          sem_in[p]).wait()

  def issue_write(g, p):
    base = wid * _PER_W + g * _CHUNK
    pltpu.async_copy(
        h_bufs[p], out_hbm.at[pl.ds(base, _CHUNK), pl.ds(0, _HD)], sem_w[p])
    pltpu.async_copy(
        r_bufs[p], out_hbm.at[pl.ds(base, _CHUNK), pl.ds(_HD, _L)], sem_w[p])

  def drain_write(p):
    pltpu.make_async_copy(
        h_bufs[p], out_hbm.at[pl.ds(0, _CHUNK), pl.ds(0, _HD)],
        sem_w[p]).wait()
    pltpu.make_async_copy(
        r_bufs[p], out_hbm.at[pl.ds(0, _CHUNK), pl.ds(_HD, _L)],
        sem_w[p]).wait()

  def half(g2, b):
    g = g2 * 2 + b
    @pl.when(g2 > 0)
    def _():
      drain_write(b)
    stage_in(g, b)
    @pl.when(g > 0)
    def _():
      drain_in(1 - b)
      issue_write(g - 1, 1 - b)

  def step(g2, carry):
    half(g2, 0)
    half(g2, 1)
    return carry

  lax.fori_loop(0, _STEPS // 2, step, 0)
  drain_in(1)
  issue_write(_STEPS - 1, 1)
  drain_write(0)
  drain_write(1)


@jax.jit
def _nlb(h, u, v, lat2d):
  mesh = plsc.VectorSubcoreMesh(
      core_axis_name="c", subcore_axis_name="s",
      num_cores=_NC, num_subcores=_NS)
  return pl.kernel(
      _body,
      out_type=jax.ShapeDtypeStruct((_N, _OD), jnp.float32),
      mesh=mesh,
      scratch_types=[
          pltpu.VMEM((_CHUNK,), jnp.float32),
          pltpu.VMEM((_CHUNK,), jnp.float32),
          pltpu.VMEM((_CHUNK,), jnp.int32),
          pltpu.VMEM((_CHUNK,), jnp.int32),
          pltpu.VMEM((_CHUNK, _HD), jnp.float32),
          pltpu.VMEM((_CHUNK, _HD), jnp.float32),
          pltpu.VMEM((_CHUNK, _L), jnp.float32),
          pltpu.VMEM((_CHUNK, _L), jnp.float32),
          pltpu.SemaphoreType.DMA,
          pltpu.SemaphoreType.DMA,
          pltpu.SemaphoreType.DMA,
          pltpu.SemaphoreType.DMA,
          pltpu.SemaphoreType.DMA,
      ],
      compiler_params=pltpu.CompilerParams(use_tc_tiling_on_sc=False),
  )(h, u, v, lat2d)


def kernel(r, h, u, v, latents):
  del r
  return _nlb(h, u, v, latents.reshape(_U * _V, _L))
